# baseline (speedup 1.0000x reference)
import functools

import jax
import jax.numpy as jnp
from jax import lax
from jax.experimental import pallas as pl
from jax.experimental.pallas import tpu as pltpu
from jax.experimental.pallas import tpu_sc as plsc

_N_SAMPLES = 512
_N_SITES = 256
_D = 512
_L = 16
_NC = 2
_NS = 16
_SPC = _N_SAMPLES // _NC
_SPT = _SPC // _NS
_RPT = _N_SITES // _NS
_NSEM = 4


def _sc_orbitals(x, table):
    mesh = plsc.VectorSubcoreMesh(core_axis_name="c", subcore_axis_name="s")

    @functools.partial(
        pl.kernel,
        out_type=jax.ShapeDtypeStruct((_N_SAMPLES * _N_SITES, _D), jnp.float32),
        mesh=mesh,
        compiler_params=pltpu.CompilerParams(needs_layout_passes=False),
        scratch_types=[
            pltpu.VMEM_SHARED((_SPC * _N_SITES,), jnp.int32),
            pltpu.VMEM((_SPT, _N_SITES), jnp.int32),
            pltpu.VMEM((_N_SITES,), jnp.int32),
            pltpu.VMEM((_N_SITES,), jnp.int32),
            pltpu.VMEM((_SPC * _N_SITES,), jnp.int32),
            pltpu.VMEM((_RPT, _D), jnp.float32),
            pltpu.SemaphoreType.DMA,
            pltpu.SemaphoreType.DMA,
            pltpu.SemaphoreType.DMA,
            pltpu.SemaphoreType.DMA,
            pltpu.SemaphoreType.DMA,
            pltpu.SemaphoreType.DMA,
            pltpu.SemaphoreType.DMA,
        ],
    )
    def k(x_hbm, tab_hbm, out_hbm, shidx, xv, rb0, rb1, lidx, res,
          s0, s1, s2, s3, rsem, p0, p1):
        cid = lax.axis_index("c")
        sid = lax.axis_index("s")
        sems = (s0, s1, s2, s3)

        rcp = pltpu.async_copy(tab_hbm.at[pl.ds(sid * _RPT, _RPT)], res, rsem)
        pltpu.sync_copy(
            x_hbm.at[pl.ds(cid * _SPC + sid * _SPT, _SPT)], xv
        )

        one_v = jnp.broadcast_to(jnp.int32(1), (_L,))

        psem = (p0, p1)
        rbufs = (rb0, rb1)

        def wait_publish(u):
            pltpu.make_async_copy(
                rbufs[u], shidx.at[pl.ds(0, _N_SITES)], psem[u]
            ).wait()

        def rank_one(t, t2, u):
            s_local = sid * _SPT + t
            s_global = cid * _SPC + s_local

            m = jnp.int32(0)
            for c in range(_N_SITES // _L):
                raw = xv[t, pl.ds(c * _L, _L)]
                m = m + jnp.sum(jnp.where(raw == one_v, one_v, one_v - one_v))
            rowbuf = rbufs[u]

            ones_cum = jnp.int32(0)
            for c in range(_N_SITES // _L):
                raw = xv[t, pl.ds(c * _L, _L)]
                occ = jnp.where(raw == one_v, one_v, one_v - one_v)
                cs = lax.cumsum(occ, axis=0)
                zcs = lax.cumsum(one_v - occ, axis=0)
                ones_off = jnp.broadcast_to(ones_cum - 1, (_L,))
                zeros_off = jnp.broadcast_to(m + (c * _L - 1) - ones_cum, (_L,))
                rank = jnp.where(occ == one_v, ones_off + cs, zeros_off + zcs)
                dest = rank + jnp.broadcast_to(s_global * _N_SITES, (_L,))
                rowbuf[pl.ds(c * _L, _L)] = dest
                ones_cum = ones_cum + jnp.sum(occ)

            pltpu.async_copy(
                rowbuf, shidx.at[pl.ds(s_local * _N_SITES, _N_SITES)], psem[u]
            )

        def do_ranks(t2, carry):
            for u in range(2):
                @pl.when(t2 > 0)
                def _():
                    wait_publish(u)
                rank_one(t2 * 2 + u, t2, u)
            return carry

        lax.fori_loop(0, _SPT // 2, do_ranks, jnp.int32(0))
        for u in range(2):
            wait_publish(u)
        plsc.subcore_barrier()

        pltpu.sync_copy(shidx, lidx)
        rcp.wait()

        def wait_scatter(j):
            dummy = lax.iota(jnp.int32, _RPT)
            pltpu.make_async_copy(res, out_hbm.at[dummy], sems[j]).wait()

        def do_scatter(t4, carry):
            for j in range(_NSEM):
                @pl.when(t4 > 0)
                def _():
                    wait_scatter(j)
                s_local = t4 * _NSEM + j
                dest = lidx[pl.ds(s_local * _N_SITES + sid * _RPT, _RPT)]
                pltpu.async_copy(res, out_hbm.at[dest], sems[j])
            return carry

        lax.fori_loop(0, _SPC // _NSEM, do_scatter, jnp.int32(0))
        for j in range(_NSEM):
            wait_scatter(j)

    return k(x, table)


def kernel(x, orbitals_mf, orbitals_hf):
    table = jnp.concatenate([orbitals_mf, orbitals_hf], axis=1)
    out = _sc_orbitals(x, table)
    return out.reshape(_N_SAMPLES, _N_SITES, _D)


# speedup vs baseline: 1.0221x; 1.0221x over previous
import functools

import jax
import jax.numpy as jnp
from jax import lax
from jax.experimental import pallas as pl
from jax.experimental.pallas import tpu as pltpu
from jax.experimental.pallas import tpu_sc as plsc

_N_SAMPLES = 512
_N_SITES = 256
_D = 512
_L = 16
_NC = 2
_NS = 16
_SPC = _N_SAMPLES // _NC
_SPT = _SPC // _NS
_RPT = _N_SITES // _NS
_NSEM = 4


def _sc_orbitals(x, table):
    mesh = plsc.VectorSubcoreMesh(core_axis_name="c", subcore_axis_name="s")

    @functools.partial(
        pl.kernel,
        out_type=jax.ShapeDtypeStruct((_N_SAMPLES * _N_SITES, _D), jnp.float32),
        mesh=mesh,
        compiler_params=pltpu.CompilerParams(needs_layout_passes=False),
        scratch_types=[
            pltpu.VMEM_SHARED((_SPC * _N_SITES,), jnp.int32),
            pltpu.VMEM((_SPT, _N_SITES), jnp.int32),
            pltpu.VMEM((_N_SITES,), jnp.int32),
            pltpu.VMEM((_N_SITES,), jnp.int32),
            pltpu.VMEM((_SPC * _N_SITES,), jnp.int32),
            pltpu.VMEM((_SPT * _RPT,), jnp.int32),
            pltpu.VMEM((_RPT, _D), jnp.float32),
            pltpu.SemaphoreType.DMA,
            pltpu.SemaphoreType.DMA,
            pltpu.SemaphoreType.DMA,
            pltpu.SemaphoreType.DMA,
            pltpu.SemaphoreType.DMA,
            pltpu.SemaphoreType.DMA,
            pltpu.SemaphoreType.DMA,
        ],
    )
    def k(x_hbm, tab_hbm, out_hbm, shidx, xv, rb0, rb1, lidx, owndest, res,
          s0, s1, s2, s3, rsem, p0, p1):
        cid = lax.axis_index("c")
        sid = lax.axis_index("s")
        sems = (s0, s1, s2, s3)

        rcp = pltpu.async_copy(tab_hbm.at[pl.ds(sid * _RPT, _RPT)], res, rsem)
        pltpu.sync_copy(
            x_hbm.at[pl.ds(cid * _SPC + sid * _SPT, _SPT)], xv
        )

        one_v = jnp.broadcast_to(jnp.int32(1), (_L,))

        psem = (p0, p1)
        rbufs = (rb0, rb1)

        def wait_publish(u):
            pltpu.make_async_copy(
                rbufs[u], shidx.at[pl.ds(0, _N_SITES)], psem[u]
            ).wait()

        def rank_one(t, t2, u):
            s_local = sid * _SPT + t
            s_global = cid * _SPC + s_local

            m = jnp.int32(0)
            for c in range(_N_SITES // _L):
                raw = xv[t, pl.ds(c * _L, _L)]
                m = m + jnp.sum(jnp.where(raw == one_v, one_v, one_v - one_v))
            rowbuf = rbufs[u]

            ones_cum = jnp.int32(0)
            for c in range(_N_SITES // _L):
                raw = xv[t, pl.ds(c * _L, _L)]
                occ = jnp.where(raw == one_v, one_v, one_v - one_v)
                cs = lax.cumsum(occ, axis=0)
                zcs = lax.cumsum(one_v - occ, axis=0)
                ones_off = jnp.broadcast_to(ones_cum - 1, (_L,))
                zeros_off = jnp.broadcast_to(m + (c * _L - 1) - ones_cum, (_L,))
                rank = jnp.where(occ == one_v, ones_off + cs, zeros_off + zcs)
                dest = rank + jnp.broadcast_to(s_global * _N_SITES, (_L,))
                rowbuf[pl.ds(c * _L, _L)] = dest
                @pl.when(sid == c)
                def _():
                    owndest[pl.ds(t * _RPT, _RPT)] = dest
                ones_cum = ones_cum + jnp.sum(occ)

            pltpu.async_copy(
                rowbuf, shidx.at[pl.ds(s_local * _N_SITES, _N_SITES)], psem[u]
            )

        def wait_scatter(j):
            dummy = lax.iota(jnp.int32, _RPT)
            pltpu.make_async_copy(res, out_hbm.at[dummy], sems[j]).wait()

        rcp.wait()

        def do_ranks(t4, carry):
            for u in range(4):
                if u < 2:
                    @pl.when(t4 > 0)
                    def _():
                        wait_publish(u)
                else:
                    wait_publish(u - 2)
                t = t4 * 4 + u
                rank_one(t, t4, u % 2)
                @pl.when(t4 > 0)
                def _():
                    wait_scatter(u)
                dest = owndest[pl.ds(t * _RPT, _RPT)]
                pltpu.async_copy(res, out_hbm.at[dest], sems[u])
            return carry

        lax.fori_loop(0, _SPT // 4, do_ranks, jnp.int32(0))
        for u in range(2):
            wait_publish(u)
        plsc.subcore_barrier()

        pltpu.sync_copy(shidx, lidx)

        def do_scatter(t4, carry):
            for j in range(_NSEM):
                wait_scatter(j)
                s_local = jnp.bitwise_and(
                    sid * _SPT + _SPT + t4 * _NSEM + j, _SPC - 1
                )
                dest = lidx[pl.ds(s_local * _N_SITES + sid * _RPT, _RPT)]
                pltpu.async_copy(res, out_hbm.at[dest], sems[j])
            return carry

        lax.fori_loop(0, (_SPC - _SPT) // _NSEM, do_scatter, jnp.int32(0))
        for j in range(_NSEM):
            wait_scatter(j)

    return k(x, table)


def kernel(x, orbitals_mf, orbitals_hf):
    table = jnp.concatenate([orbitals_mf, orbitals_hf], axis=1)
    out = _sc_orbitals(x, table)
    return out.reshape(_N_SAMPLES, _N_SITES, _D)
